# baseline (speedup 1.0000x reference)
import jax
import jax.numpy as jnp
from jax import lax
from jax.experimental import pallas as pl
from jax.experimental.pallas import tpu as pltpu
from jax.experimental.pallas import tpu_sc as plsc

N = 10000
E = 160000
D = 256
DE = 16
HD = D // 2

NC, NS = 2, 16
NW = NC * NS
CH = 128
E_PAD = 163840
N_PAD = 10240
EPW = E_PAD // NW
CPW = EPW // CH
EPT = E_PAD // NS
CPT = EPT // CH
RPT = N_PAD // NS
ZK = RPT // CH

QD = D // 4

DUMMY = N

_mesh = plsc.VectorSubcoreMesh(
    core_axis_name="c", subcore_axis_name="s", num_cores=NC, num_subcores=NS)


def _wid():
    return lax.axis_index("s") * NC + lax.axis_index("c")


NBUF = 8
GCH = 64
GPW = EPW // GCH
GP0 = 80
GP1 = 2 * GPW - GP0
GPMAX = max(GP0, GP1)


def _gather_body(tab, src2, out, idx_v, bufs, gsem, wsem):
    c = lax.axis_index("c")
    s = lax.axis_index("s")
    base_c = jnp.where(c == 0, s * GP0, NS * GP0 + s * GP1)
    n_c = jnp.where(c == 0, GP0, GP1)

    @pl.when(c == 0)
    def _():
        pltpu.sync_copy(src2.at[pl.ds(base_c, GP0)], idx_v.at[pl.ds(0, GP0)])

    @pl.when(c == 1)
    def _():
        pltpu.sync_copy(src2.at[pl.ds(base_c, GP1)], idx_v.at[pl.ds(0, GP1)])

    for b in range(NBUF):
        pltpu.async_copy(tab.at[idx_v.at[b]], bufs[b], gsem[b])

    def grp(j8, _):
        base = j8 * NBUF
        for b in range(NBUF):
            pltpu.make_async_copy(
                tab.at[pl.ds(0, GCH)], bufs[b], gsem[b]).wait()
            pltpu.async_copy(
                bufs[b], out.at[pl.ds((base_c + base + b) * GCH, GCH)],
                wsem[b])
        for b in range(NBUF):
            nxt = base + NBUF + b

            @pl.when(nxt < n_c)
            def _():
                pltpu.make_async_copy(
                    bufs[b], out.at[pl.ds(base_c * GCH, GCH)], wsem[b]).wait()
                pltpu.async_copy(tab.at[idx_v.at[nxt]], bufs[b], gsem[b])
        return 0

    lax.fori_loop(0, n_c // NBUF, grp, 0)
    for b in range(NBUF):
        pltpu.make_async_copy(
            bufs[b], out.at[pl.ds(base_c * GCH, GCH)], wsem[b]).wait()


def _gather_entry(tab, src2, out, idx_v, *rest):
    bufs = rest[:NBUF]
    gsem = rest[NBUF:2 * NBUF]
    wsem = rest[2 * NBUF:3 * NBUF]
    _gather_body(tab, src2, out, idx_v, bufs, gsem, wsem)


_sc_gather = pl.kernel(
    _gather_entry,
    out_type=jax.ShapeDtypeStruct((E_PAD, HD), jnp.int32),
    mesh=_mesh,
    scratch_types=(
        [pltpu.VMEM((GPMAX, GCH), jnp.int32)]
        + [pltpu.VMEM((GCH, HD), jnp.int32)] * NBUF
        + [pltpu.SemaphoreType.DMA] * (2 * NBUF)
    ),
)


VL = 16


def _cnt_body(dst_f, out, idx_v, cnt_v):
    w = _wid()

    def zstep(i, _):
        cnt_v[pl.ds(i * VL, VL)] = jnp.zeros((VL,), jnp.float32)
        return 0

    lax.fori_loop(0, N_PAD // VL, zstep, 0)
    pltpu.sync_copy(dst_f.at[pl.ds(w * EPW, EPW)], idx_v)
    ones = jnp.ones((VL,), jnp.float32)

    def step(i, _):
        idx16 = idx_v[pl.ds(i * VL, VL)]
        plsc.addupdate_scatter(cnt_v, [idx16], ones)
        return 0

    lax.fori_loop(0, EPW // VL, step, 0)
    pltpu.sync_copy(cnt_v, out.at[w])


_sc_cnt = pl.kernel(
    _cnt_body,
    out_type=jax.ShapeDtypeStruct((NW, N_PAD), jnp.float32),
    mesh=_mesh,
    scratch_types=[
        pltpu.VMEM((EPW,), jnp.int32),
        pltpu.VMEM((N_PAD,), jnp.float32),
    ],
    compiler_params=pltpu.CompilerParams(needs_layout_passes=False),
)


def _cnt_reduce_kernel(cp, o):
    o[...] = jnp.sum(cp[...], axis=0)[:, None]


BC = 1024


def _tc_cnt_reduce(cp):
    return pl.pallas_call(
        _cnt_reduce_kernel,
        grid=(N_PAD // BC,),
        in_specs=[pl.BlockSpec((NW, BC), lambda i: (0, i))],
        out_specs=pl.BlockSpec((BC, 1), lambda i: (i, 0)),
        out_shape=jax.ShapeDtypeStruct((N_PAD, 1), jnp.float32),
    )(cp)


NB = 4


def _scatter_body(m2s, dst2, zrows, accs,
                  idx_v, zbuf, m0, m1, m2, m3, l0, l1, l2, l3, acc_sh):
    c = lax.axis_index("c")
    s = lax.axis_index("s")
    mbufs = (m0, m1, m2, m3)
    lsem = (l0, l1, l2, l3)

    pltpu.sync_copy(zrows, zbuf)
    for k in range(ZK):
        pltpu.sync_copy(zbuf, acc_sh.at[pl.ds(s * RPT + k * CH, CH)])
    pltpu.sync_copy(dst2.at[pl.ds(s * CPT, CPT)], idx_v)
    plsc.subcore_barrier()

    def step(j, _):
        pltpu.sync_copy(m2s.at[c, pl.ds((s * CPT + j) * CH, CH)], mbufs[0])
        pltpu.sync_copy(mbufs[0], acc_sh.at[idx_v.at[j]], add=True)
        return 0

    lax.fori_loop(0, CPT, step, 0)
    plsc.subcore_barrier()
    for k in range(ZK):
        pltpu.sync_copy(acc_sh.at[pl.ds(s * RPT + k * CH, CH)], zbuf)
        pltpu.sync_copy(zbuf, accs.at[c, pl.ds(s * RPT + k * CH, CH)])


_sc_scatter = pl.kernel(
    _scatter_body,
    out_type=jax.ShapeDtypeStruct((NC, N_PAD, HD), jnp.float32),
    mesh=_mesh,
    scratch_types=[
        pltpu.VMEM((CPT, CH), jnp.int32),
        pltpu.VMEM((CH, HD), jnp.float32),
        pltpu.VMEM((CH, HD), jnp.float32),
        pltpu.VMEM((CH, HD), jnp.float32),
        pltpu.VMEM((CH, HD), jnp.float32),
        pltpu.VMEM((CH, HD), jnp.float32),
        pltpu.SemaphoreType.DMA,
        pltpu.SemaphoreType.DMA,
        pltpu.SemaphoreType.DMA,
        pltpu.SemaphoreType.DMA,
        pltpu.VMEM_SHARED((N_PAD, HD), jnp.float32),
    ],
)


BE = 2048
BN = 1000


def _pack_bf16_pair(lo_f32, hi_f32):
    lo = lax.bitcast_convert_type(lo_f32.astype(jnp.bfloat16), jnp.uint16)
    hi = lax.bitcast_convert_type(hi_f32.astype(jnp.bfloat16), jnp.uint16)
    word = lo.astype(jnp.uint32) | (hi.astype(jnp.uint32) << 16)
    return lax.bitcast_convert_type(word, jnp.int32)


def _unpack_bf16_pair(word_i32):
    w = lax.bitcast_convert_type(word_i32, jnp.uint32)
    lo = lax.bitcast_convert_type((w & 0xFFFF).astype(jnp.uint16), jnp.bfloat16)
    hi = lax.bitcast_convert_type((w >> 16).astype(jnp.uint16), jnp.bfloat16)
    return lo, hi


def _lin_kernel(h, w, b, o):
    a = jnp.dot(h[...].astype(jnp.bfloat16), w[...],
                preferred_element_type=jnp.float32) + b[...]
    o[...] = _pack_bf16_pair(a[:, :HD], a[:, HD:])


def _tc_linear(h, w, b):
    return pl.pallas_call(
        _lin_kernel,
        grid=(N // BN,),
        in_specs=[
            pl.BlockSpec((BN, D), lambda i: (i, 0)),
            pl.BlockSpec((D, D), lambda i: (0, 0)),
            pl.BlockSpec((1, D), lambda i: (0, 0)),
        ],
        out_specs=pl.BlockSpec((BN, HD), lambda i: (i, 0)),
        out_shape=jax.ShapeDtypeStruct((N, HD), jnp.int32),
    )(h, w, b)


def _edge_kernel(g, ea, w0b, w1, b1, o):
    ew = jnp.dot(ea[...], w0b[...], preferred_element_type=jnp.float32)
    glo, ghi = _unpack_bf16_pair(g[...])
    m1lo = jnp.maximum(glo.astype(jnp.float32) + ew[:, :HD], 0.0)
    m1hi = jnp.maximum(ghi.astype(jnp.float32) + ew[:, HD:], 0.0)
    m2 = jnp.dot(m1lo.astype(jnp.bfloat16), w1[:HD, :],
                 preferred_element_type=jnp.float32)
    m2 += jnp.dot(m1hi.astype(jnp.bfloat16), w1[HD:, :],
                  preferred_element_type=jnp.float32)
    m2 = jnp.maximum(m2 + b1[...], 0.0)
    o[0] = m2[:, :HD]
    o[1] = m2[:, HD:]


def _tc_edge(g, ea, w0b, w1, b1):
    return pl.pallas_call(
        _edge_kernel,
        grid=(E_PAD // BE,),
        in_specs=[
            pl.BlockSpec((BE, HD), lambda i: (i, 0)),
            pl.BlockSpec((BE, DE), lambda i: (i, 0)),
            pl.BlockSpec((DE, D), lambda i: (0, 0)),
            pl.BlockSpec((D, D), lambda i: (0, 0)),
            pl.BlockSpec((1, D), lambda i: (0, 0)),
        ],
        out_specs=pl.BlockSpec((NC, BE, HD), lambda i: (0, i, 0)),
        out_shape=jax.ShapeDtypeStruct((NC, E_PAD, HD), jnp.float32),
    )(g, ea, w0b, w1, b1)


def _node_kernel(h, aa, ab, c0, wr0a, wr0b, br0, wr1, br1, o):
    c = jnp.maximum(c0[...], 1.0)
    recip = 1.0 / c
    r = jnp.dot(h[...].astype(jnp.bfloat16), wr0a[...],
                preferred_element_type=jnp.float32)
    r += jnp.dot((aa[0] * recip).astype(jnp.bfloat16), wr0b[:HD, :],
                 preferred_element_type=jnp.float32)
    r += jnp.dot((ab[0] * recip).astype(jnp.bfloat16), wr0b[HD:, :],
                 preferred_element_type=jnp.float32)
    r = jnp.maximum(r + br0[...], 0.0)
    o[...] = jnp.dot(r.astype(jnp.bfloat16), wr1[...],
                     preferred_element_type=jnp.float32) + br1[...]


def _tc_node(h, aa, ab, c0, wr0a, wr0b, br0, wr1, br1):
    return pl.pallas_call(
        _node_kernel,
        grid=(N // BN,),
        in_specs=[
            pl.BlockSpec((BN, D), lambda i: (i, 0)),
            pl.BlockSpec((1, BN, HD), lambda i: (0, i, 0)),
            pl.BlockSpec((1, BN, HD), lambda i: (1, i, 0)),
            pl.BlockSpec((BN, 1), lambda i: (i, 0)),
            pl.BlockSpec((D, D), lambda i: (0, 0)),
            pl.BlockSpec((D, D), lambda i: (0, 0)),
            pl.BlockSpec((1, D), lambda i: (0, 0)),
            pl.BlockSpec((D, D), lambda i: (0, 0)),
            pl.BlockSpec((1, D), lambda i: (0, 0)),
        ],
        out_specs=pl.BlockSpec((BN, D), lambda i: (i, 0)),
        out_shape=jax.ShapeDtypeStruct((N, D), jnp.float32),
    )(h, aa, ab, c0, wr0a, wr0b, br0, wr1, br1)


def kernel(x, edge_index, edge_attr, Wm0, bm0, Wm1, bm1, Wr0, br0, Wr1, br1):
    L = Wm0.shape[0]
    src = edge_index[0].astype(jnp.int32)
    dst = edge_index[1].astype(jnp.int32)
    pad = E_PAD - E
    src_p = jnp.concatenate([src, jnp.zeros((pad,), jnp.int32)])
    dst_p = jnp.concatenate([dst, jnp.full((pad,), DUMMY, jnp.int32)])
    ea_p = jnp.concatenate([edge_attr,
                            jnp.zeros((pad, DE), edge_attr.dtype)], axis=0)
    src2 = src_p.reshape(E_PAD // GCH, GCH)
    dst2 = dst_p.reshape(E_PAD // CH, CH)

    zrows = jnp.zeros((CH, HD), jnp.float32)

    cnt_parts = _sc_cnt(dst_p)
    c0 = _tc_cnt_reduce(cnt_parts)[:N]

    bf = jnp.bfloat16
    h = x
    for i in range(L):
        a = _tc_linear(h, Wm0[i, :D, :].astype(bf), bm0[i][None, :])
        g = _sc_gather(a, src2)
        m2s = _tc_edge(g, ea_p, Wm0[i, D:, :], Wm1[i].astype(bf),
                       bm1[i][None, :])
        accs = _sc_scatter(m2s, dst2, zrows)
        h = _tc_node(h, accs, accs, c0,
                     Wr0[i, :D, :].astype(bf), Wr0[i, D:, :].astype(bf),
                     br0[i][None, :], Wr1[i].astype(bf), br1[i][None, :])
    return h


# speedup vs baseline: 1.0287x; 1.0287x over previous
import jax
import jax.numpy as jnp
from jax import lax
from jax.experimental import pallas as pl
from jax.experimental.pallas import tpu as pltpu
from jax.experimental.pallas import tpu_sc as plsc

N = 10000
E = 160000
D = 256
DE = 16
HD = D // 2

NC, NS = 2, 16
NW = NC * NS
CH = 128
E_PAD = 163840
N_PAD = 10240
EPW = E_PAD // NW
CPW = EPW // CH
EPT = E_PAD // NS
CPT = EPT // CH
RPT = N_PAD // NS
ZK = RPT // CH

QD = D // 4

DUMMY = N

_mesh = plsc.VectorSubcoreMesh(
    core_axis_name="c", subcore_axis_name="s", num_cores=NC, num_subcores=NS)


def _wid():
    return lax.axis_index("s") * NC + lax.axis_index("c")


NBUF = 8
GCH = 64
GPW = EPW // GCH
GP0 = 80
GP1 = 2 * GPW - GP0
GPMAX = max(GP0, GP1)


def _gather_body(tab, src2, out, idx_v, bufs, gsem, wsem):
    c = lax.axis_index("c")
    s = lax.axis_index("s")
    base_c = jnp.where(c == 0, s * GP0, NS * GP0 + s * GP1)
    n_c = jnp.where(c == 0, GP0, GP1)

    @pl.when(c == 0)
    def _():
        pltpu.sync_copy(src2.at[pl.ds(base_c, GP0)], idx_v.at[pl.ds(0, GP0)])

    @pl.when(c == 1)
    def _():
        pltpu.sync_copy(src2.at[pl.ds(base_c, GP1)], idx_v.at[pl.ds(0, GP1)])

    for b in range(NBUF):
        pltpu.async_copy(tab.at[idx_v.at[b]], bufs[b], gsem[b])

    def grp(j8, _):
        base = j8 * NBUF
        for b in range(NBUF):
            pltpu.make_async_copy(
                tab.at[pl.ds(0, GCH)], bufs[b], gsem[b]).wait()
            pltpu.async_copy(
                bufs[b], out.at[pl.ds((base_c + base + b) * GCH, GCH)],
                wsem[b])
        for b in range(NBUF):
            nxt = base + NBUF + b

            @pl.when(nxt < n_c)
            def _():
                pltpu.make_async_copy(
                    bufs[b], out.at[pl.ds(base_c * GCH, GCH)], wsem[b]).wait()
                pltpu.async_copy(tab.at[idx_v.at[nxt]], bufs[b], gsem[b])
        return 0

    lax.fori_loop(0, n_c // NBUF, grp, 0)
    for b in range(NBUF):
        pltpu.make_async_copy(
            bufs[b], out.at[pl.ds(base_c * GCH, GCH)], wsem[b]).wait()


def _gather_entry(tab, src2, out, idx_v, *rest):
    bufs = rest[:NBUF]
    gsem = rest[NBUF:2 * NBUF]
    wsem = rest[2 * NBUF:3 * NBUF]
    _gather_body(tab, src2, out, idx_v, bufs, gsem, wsem)


_sc_gather = pl.kernel(
    _gather_entry,
    out_type=jax.ShapeDtypeStruct((E_PAD, HD), jnp.int32),
    mesh=_mesh,
    scratch_types=(
        [pltpu.VMEM((GPMAX, GCH), jnp.int32)]
        + [pltpu.VMEM((GCH, HD), jnp.int32)] * NBUF
        + [pltpu.SemaphoreType.DMA] * (2 * NBUF)
    ),
)


def _cnt_body(dst2, zrow, onerow, out0, out1, idx_v, one_v, stage_v, cnt_sh):
    c = lax.axis_index("c")
    s = lax.axis_index("s")
    w = _wid()
    pltpu.sync_copy(zrow, stage_v)
    pltpu.sync_copy(stage_v, cnt_sh.at[pl.ds(s * RPT, RPT)])
    pltpu.sync_copy(onerow, one_v)
    pltpu.sync_copy(dst2.at[pl.ds(w * CPW, CPW)], idx_v)
    plsc.subcore_barrier()

    def step(j, _):
        pltpu.sync_copy(one_v, cnt_sh.at[idx_v.at[j]], add=True)
        return 0

    lax.fori_loop(0, CPW, step, 0)
    plsc.subcore_barrier()
    pltpu.sync_copy(cnt_sh.at[pl.ds(s * RPT, RPT)], stage_v)

    @pl.when(c == 0)
    def _():
        pltpu.sync_copy(stage_v, out0.at[pl.ds(s * RPT, RPT)])

    @pl.when(c == 1)
    def _():
        pltpu.sync_copy(stage_v, out1.at[pl.ds(s * RPT, RPT)])


_sc_cnt = pl.kernel(
    _cnt_body,
    out_type=(jax.ShapeDtypeStruct((N_PAD,), jnp.float32),
              jax.ShapeDtypeStruct((N_PAD,), jnp.float32)),
    mesh=_mesh,
    scratch_types=[
        pltpu.VMEM((CPW, CH), jnp.int32),
        pltpu.VMEM((CH,), jnp.float32),
        pltpu.VMEM((RPT,), jnp.float32),
        pltpu.VMEM_SHARED((N_PAD,), jnp.float32),
    ],
)


NB = 4


def _scatter_body(m2s, dst2, zrows, accs,
                  idx_v, zbuf, m0, m1, m2, m3, l0, l1, l2, l3, acc_sh):
    c = lax.axis_index("c")
    s = lax.axis_index("s")
    mbufs = (m0, m1, m2, m3)
    lsem = (l0, l1, l2, l3)

    pltpu.sync_copy(zrows, zbuf)
    for k in range(ZK):
        pltpu.sync_copy(zbuf, acc_sh.at[pl.ds(s * RPT + k * CH, CH)])
    pltpu.sync_copy(dst2.at[pl.ds(s * CPT, CPT)], idx_v)
    plsc.subcore_barrier()

    def step(j, _):
        pltpu.sync_copy(m2s.at[c, pl.ds((s * CPT + j) * CH, CH)], mbufs[0])
        pltpu.sync_copy(mbufs[0], acc_sh.at[idx_v.at[j]], add=True)
        return 0

    lax.fori_loop(0, CPT, step, 0)
    plsc.subcore_barrier()
    for k in range(ZK):
        pltpu.sync_copy(acc_sh.at[pl.ds(s * RPT + k * CH, CH)], zbuf)
        pltpu.sync_copy(zbuf, accs.at[c, pl.ds(s * RPT + k * CH, CH)])


_sc_scatter = pl.kernel(
    _scatter_body,
    out_type=jax.ShapeDtypeStruct((NC, N_PAD, HD), jnp.float32),
    mesh=_mesh,
    scratch_types=[
        pltpu.VMEM((CPT, CH), jnp.int32),
        pltpu.VMEM((CH, HD), jnp.float32),
        pltpu.VMEM((CH, HD), jnp.float32),
        pltpu.VMEM((CH, HD), jnp.float32),
        pltpu.VMEM((CH, HD), jnp.float32),
        pltpu.VMEM((CH, HD), jnp.float32),
        pltpu.SemaphoreType.DMA,
        pltpu.SemaphoreType.DMA,
        pltpu.SemaphoreType.DMA,
        pltpu.SemaphoreType.DMA,
        pltpu.VMEM_SHARED((N_PAD, HD), jnp.float32),
    ],
)


BE = 2048
BN = 1000


def _pack_bf16_pair(lo_f32, hi_f32):
    lo = lax.bitcast_convert_type(lo_f32.astype(jnp.bfloat16), jnp.uint16)
    hi = lax.bitcast_convert_type(hi_f32.astype(jnp.bfloat16), jnp.uint16)
    word = lo.astype(jnp.uint32) | (hi.astype(jnp.uint32) << 16)
    return lax.bitcast_convert_type(word, jnp.int32)


def _unpack_bf16_pair(word_i32):
    w = lax.bitcast_convert_type(word_i32, jnp.uint32)
    lo = lax.bitcast_convert_type((w & 0xFFFF).astype(jnp.uint16), jnp.bfloat16)
    hi = lax.bitcast_convert_type((w >> 16).astype(jnp.uint16), jnp.bfloat16)
    return lo, hi


def _lin_kernel(h, w, b, o):
    a = jnp.dot(h[...].astype(jnp.bfloat16), w[...],
                preferred_element_type=jnp.float32) + b[...]
    o[...] = _pack_bf16_pair(a[:, :HD], a[:, HD:])


def _tc_linear(h, w, b):
    return pl.pallas_call(
        _lin_kernel,
        grid=(N // BN,),
        in_specs=[
            pl.BlockSpec((BN, D), lambda i: (i, 0)),
            pl.BlockSpec((D, D), lambda i: (0, 0)),
            pl.BlockSpec((1, D), lambda i: (0, 0)),
        ],
        out_specs=pl.BlockSpec((BN, HD), lambda i: (i, 0)),
        out_shape=jax.ShapeDtypeStruct((N, HD), jnp.int32),
    )(h, w, b)


def _edge_kernel(g, ea, w0b, w1, b1, o):
    ew = jnp.dot(ea[...], w0b[...], preferred_element_type=jnp.float32)
    glo, ghi = _unpack_bf16_pair(g[...])
    m1lo = jnp.maximum(glo.astype(jnp.float32) + ew[:, :HD], 0.0)
    m1hi = jnp.maximum(ghi.astype(jnp.float32) + ew[:, HD:], 0.0)
    m2 = jnp.dot(m1lo.astype(jnp.bfloat16), w1[:HD, :],
                 preferred_element_type=jnp.float32)
    m2 += jnp.dot(m1hi.astype(jnp.bfloat16), w1[HD:, :],
                  preferred_element_type=jnp.float32)
    m2 = jnp.maximum(m2 + b1[...], 0.0)
    o[0] = m2[:, :HD]
    o[1] = m2[:, HD:]


def _tc_edge(g, ea, w0b, w1, b1):
    return pl.pallas_call(
        _edge_kernel,
        grid=(E_PAD // BE,),
        in_specs=[
            pl.BlockSpec((BE, HD), lambda i: (i, 0)),
            pl.BlockSpec((BE, DE), lambda i: (i, 0)),
            pl.BlockSpec((DE, D), lambda i: (0, 0)),
            pl.BlockSpec((D, D), lambda i: (0, 0)),
            pl.BlockSpec((1, D), lambda i: (0, 0)),
        ],
        out_specs=pl.BlockSpec((NC, BE, HD), lambda i: (0, i, 0)),
        out_shape=jax.ShapeDtypeStruct((NC, E_PAD, HD), jnp.float32),
    )(g, ea, w0b, w1, b1)


def _node_kernel(h, aa, ab, c0, c1, wr0a, wr0b, br0, wr1, br1, o):
    c = jnp.maximum(c0[...] + c1[...], 1.0)
    recip = 1.0 / c
    r = jnp.dot(h[...].astype(jnp.bfloat16), wr0a[...],
                preferred_element_type=jnp.float32)
    r += jnp.dot((aa[0] * recip).astype(jnp.bfloat16), wr0b[:HD, :],
                 preferred_element_type=jnp.float32)
    r += jnp.dot((ab[0] * recip).astype(jnp.bfloat16), wr0b[HD:, :],
                 preferred_element_type=jnp.float32)
    r = jnp.maximum(r + br0[...], 0.0)
    o[...] = jnp.dot(r.astype(jnp.bfloat16), wr1[...],
                     preferred_element_type=jnp.float32) + br1[...]


def _tc_node(h, aa, ab, c0, c1, wr0a, wr0b, br0, wr1, br1):
    return pl.pallas_call(
        _node_kernel,
        grid=(N // BN,),
        in_specs=[
            pl.BlockSpec((BN, D), lambda i: (i, 0)),
            pl.BlockSpec((1, BN, HD), lambda i: (0, i, 0)),
            pl.BlockSpec((1, BN, HD), lambda i: (1, i, 0)),
            pl.BlockSpec((BN, 1), lambda i: (i, 0)),
            pl.BlockSpec((BN, 1), lambda i: (i, 0)),
            pl.BlockSpec((D, D), lambda i: (0, 0)),
            pl.BlockSpec((D, D), lambda i: (0, 0)),
            pl.BlockSpec((1, D), lambda i: (0, 0)),
            pl.BlockSpec((D, D), lambda i: (0, 0)),
            pl.BlockSpec((1, D), lambda i: (0, 0)),
        ],
        out_specs=pl.BlockSpec((BN, D), lambda i: (i, 0)),
        out_shape=jax.ShapeDtypeStruct((N, D), jnp.float32),
    )(h, aa, ab, c0, c1, wr0a, wr0b, br0, wr1, br1)


def kernel(x, edge_index, edge_attr, Wm0, bm0, Wm1, bm1, Wr0, br0, Wr1, br1):
    L = Wm0.shape[0]
    src = edge_index[0].astype(jnp.int32)
    dst = edge_index[1].astype(jnp.int32)
    pad = E_PAD - E
    src_p = jnp.concatenate([src, jnp.zeros((pad,), jnp.int32)])
    dst_p = jnp.concatenate([dst, jnp.full((pad,), DUMMY, jnp.int32)])
    ea_p = jnp.concatenate([edge_attr,
                            jnp.zeros((pad, DE), edge_attr.dtype)], axis=0)
    src2 = src_p.reshape(E_PAD // GCH, GCH)
    dst2 = dst_p.reshape(E_PAD // CH, CH)

    zrows = jnp.zeros((CH, HD), jnp.float32)
    zrow = jnp.zeros((RPT,), jnp.float32)
    onerow = jnp.ones((CH,), jnp.float32)

    cnt0, cnt1 = _sc_cnt(dst2, zrow, onerow)
    c0 = cnt0[:N, None]
    c1 = cnt1[:N, None]

    bf = jnp.bfloat16
    h = x
    for i in range(L):
        a = _tc_linear(h, Wm0[i, :D, :].astype(bf), bm0[i][None, :])
        g = _sc_gather(a, src2)
        m2s = _tc_edge(g, ea_p, Wm0[i, D:, :], Wm1[i].astype(bf),
                       bm1[i][None, :])
        accs = _sc_scatter(m2s, dst2, zrows)
        h = _tc_node(h, accs, accs, c0, c1,
                     Wr0[i, :D, :].astype(bf), Wr0[i, D:, :].astype(bf),
                     br0[i][None, :], Wr1[i].astype(bf), br1[i][None, :])
    return h
